# baseline (speedup 1.0000x reference)
import jax
import jax.numpy as jnp
from jax import lax
from jax.experimental import pallas as pl
from jax.experimental.pallas import tpu as pltpu
from jax.experimental.pallas import tpu_sc as plsc

_L = 16
_C = 128
_G = _C // _L
_K = 100
_SCORE_THRESH = 0.5
_NMS_THRESH = 0.5


def _nms_body(x1h, y1h, x2h, y2h, sch, lbh, order_hbm, out_hbm,
              idx_v, cx1, cy1, cx2, cy2, csc, clb, car, alive_v,
              kx1, ky1, kx2, ky2, ksc, klb, kar, sem):
    n = sch.shape[0]
    n_pad = order_hbm.shape[0]
    n_chunks = n_pad // _C
    wid = lax.axis_index("s") * 2 + lax.axis_index("c")

    @pl.when(wid == 0)
    def _run():
        lane = lax.broadcasted_iota(jnp.int32, (_L,), 0)
        zero = jnp.zeros((_L,), jnp.float32)

        alive_v[pl.ds(_C, _L)] = jnp.zeros((_L,), jnp.int32)

        for g in range(_G):
            sl = pl.ds(g * _L, _L)
            kx1[sl] = zero
            ky1[sl] = zero
            kx2[sl] = zero
            ky2[sl] = zero
            ksc[sl] = zero
            klb[sl] = zero

        def splat_at(vec, j):
            return vec.at[jnp.full((_L,), j, jnp.int32)].get(
                mode="promise_in_bounds")

        def store_at(ref, pos, val_splat):
            gb = pos & ~(_L - 1)
            grp = ref[pl.ds(gb, _L)]
            ref[pl.ds(gb, _L)] = jnp.where(lane == pos - gb, val_splat, grp)

        def iou_mask(bx1, by1, bx2, by2, ba, sl):
            xx1 = jnp.maximum(cx1[sl], bx1)
            yy1 = jnp.maximum(cy1[sl], by1)
            xx2 = jnp.minimum(cx2[sl], bx2)
            yy2 = jnp.minimum(cy2[sl], by2)
            inter = (jnp.maximum(xx2 - xx1, 0.0)
                     * jnp.maximum(yy2 - yy1, 0.0))
            iou = inter / (ba + car[sl] - inter + 1e-9)
            return iou > _NMS_THRESH

        def load_chunk(chunk):
            pltpu.sync_copy(order_hbm.at[pl.ds(chunk * _C, _C)], idx_v)
            cps = [pltpu.async_copy(h.at[idx_v], v, sem)
                   for h, v in ((x1h, cx1), (y1h, cy1), (x2h, cx2),
                                (y2h, cy2), (sch, csc), (lbh, clb))]
            for cp in cps:
                cp.wait()
            for g in range(_G):
                sl = pl.ds(g * _L, _L)
                car[sl] = (cx2[sl] - cx1[sl]) * (cy2[sl] - cy1[sl])
                alive_v[sl] = jnp.where(
                    (csc[sl] >= _SCORE_THRESH)
                    & (lane < n - (chunk * _C + g * _L)), 1, 0)

        def phase1(count):
            def body(k, _):
                gb = k & ~(_L - 1)
                lidx = k - gb
                sl = pl.ds(gb, _L)
                b1 = splat_at(kx1[sl], lidx)
                b2 = splat_at(ky1[sl], lidx)
                b3 = splat_at(kx2[sl], lidx)
                b4 = splat_at(ky2[sl], lidx)
                ba = splat_at(kar[sl], lidx)
                for g in range(_G):
                    gsl = pl.ds(g * _L, _L)
                    alive_v[gsl] = jnp.where(
                        iou_mask(b1, b2, b3, b4, ba, gsl), 0, alive_v[gsl])
                return 0
            lax.fori_loop(0, count, body, 0)

        def phase2(count):
            def body(p, cnt):
                gb = p & ~(_L - 1)
                lidx = p - gb
                sl = pl.ds(gb, _L)
                a = alive_v[pl.ds(p, _L)][0]

                def keep_it(cnt):
                    b1 = splat_at(cx1[sl], lidx)
                    b2 = splat_at(cy1[sl], lidx)
                    b3 = splat_at(cx2[sl], lidx)
                    b4 = splat_at(cy2[sl], lidx)
                    ba = splat_at(car[sl], lidx)
                    store_at(kx1, cnt, b1)
                    store_at(ky1, cnt, b2)
                    store_at(kx2, cnt, b3)
                    store_at(ky2, cnt, b4)
                    store_at(kar, cnt, ba)
                    store_at(ksc, cnt, splat_at(csc[sl], lidx))
                    store_at(klb, cnt, splat_at(clb[sl], lidx))
                    def supp(g, _):
                        gsl = pl.ds(g * _L, _L)
                        alive_v[gsl] = jnp.where(
                            iou_mask(b1, b2, b3, b4, ba, gsl),
                            0, alive_v[gsl])
                        return 0
                    lax.fori_loop(p // _L, _G, supp, 0)
                    return cnt + 1

                return lax.cond((a != 0) & (cnt < _K), keep_it,
                                lambda cnt: cnt, cnt)

            return lax.fori_loop(0, _C, body, count)

        def outer_body(chunk, carry):
            def process(carry):
                count, more = carry
                load_chunk(chunk)
                sc_top = csc[pl.ds(0, _L)][0]

                @pl.when((sc_top >= _SCORE_THRESH) & (count > 0))
                def _p1():
                    phase1(count)

                count = lax.cond(sc_top >= _SCORE_THRESH, phase2,
                                 lambda cnt: cnt, count)
                more = (csc[pl.ds(_C - _L, _L)][_L - 1]
                        >= _SCORE_THRESH).astype(jnp.int32)
                return (count, more)

            count, more = carry
            active = (more != 0) & (count < _K)
            return lax.cond(active, process, lambda c: c, carry)

        lax.fori_loop(0, n_chunks, outer_body,
                      (jnp.int32(0), jnp.int32(1)))

        pltpu.sync_copy(kx1, out_hbm.at[0])
        pltpu.sync_copy(ky1, out_hbm.at[1])
        pltpu.sync_copy(kx2, out_hbm.at[2])
        pltpu.sync_copy(ky2, out_hbm.at[3])
        pltpu.sync_copy(ksc, out_hbm.at[4])
        pltpu.sync_copy(klb, out_hbm.at[5])


def _build_nms():
    mesh = plsc.VectorSubcoreMesh(core_axis_name="c", subcore_axis_name="s")
    col = pltpu.VMEM((_C,), jnp.float32)
    return pl.kernel(
        _nms_body,
        out_type=jax.ShapeDtypeStruct((6, _C), jnp.float32),
        mesh=mesh,
        scratch_types=[
            pltpu.VMEM((_C,), jnp.int32),
            col, col, col, col, col, col,
            col,
            pltpu.VMEM((_C + _L,), jnp.int32),
            col, col, col, col, col, col,
            col,
            pltpu.SemaphoreType.DMA,
        ],
    )


@jax.jit
def kernel(boxes, scores, labels, top_k):
    n = boxes.shape[0]
    order = jnp.argsort(-scores).astype(jnp.int32)
    pad = (-n) % _C
    order_p = jnp.concatenate([order, jnp.zeros((pad,), jnp.int32)])
    bt = boxes.T
    lb = labels.astype(jnp.float32)

    rows = _build_nms()(bt[0], bt[1], bt[2], bt[3], scores, lb, order_p)

    out_scores = rows[4, :_K]
    valid = (out_scores > 0.0) & (jnp.arange(_K) < top_k)
    out_boxes = jnp.where(valid[:, None], rows[:4, :_K].T, 0.0)
    out_scores = jnp.where(valid, out_scores, 0.0)
    out_labels = jnp.where(valid, rows[5, :_K].astype(jnp.int32), -1)
    return out_boxes, out_scores, out_labels


# speedup vs baseline: 1.1037x; 1.1037x over previous
import jax
import jax.numpy as jnp
from jax import lax
from jax.experimental import pallas as pl
from jax.experimental.pallas import tpu as pltpu
from jax.experimental.pallas import tpu_sc as plsc

_L = 16
_C = 128
_G = _C // _L
_K = 100
_SCORE_THRESH = 0.5
_NMS_THRESH = 0.5


def _nms_body(x1h, y1h, x2h, y2h, sch, lbh, order_hbm, out_hbm,
              idx_v, cx1, cy1, cx2, cy2, csc, clb, car, alive_v,
              kx1, ky1, kx2, ky2, ksc, klb, kar, sem):
    n = sch.shape[0]
    n_pad = order_hbm.shape[0]
    n_chunks = n_pad // _C
    wid = lax.axis_index("s") * 2 + lax.axis_index("c")

    @pl.when(wid == 0)
    def _run():
        lane = lax.broadcasted_iota(jnp.int32, (_L,), 0)
        zero = jnp.zeros((_L,), jnp.float32)

        alive_v[pl.ds(_C, _L)] = jnp.zeros((_L,), jnp.int32)

        for g in range(_G):
            sl = pl.ds(g * _L, _L)
            kx1[sl] = zero
            ky1[sl] = zero
            kx2[sl] = zero
            ky2[sl] = zero
            ksc[sl] = zero
            klb[sl] = zero

        def splat_at(vec, j):
            return vec.at[jnp.full((_L,), j, jnp.int32)].get(
                mode="promise_in_bounds")

        def store_at(ref, pos, val_splat):
            gb = pos & ~(_L - 1)
            grp = ref[pl.ds(gb, _L)]
            ref[pl.ds(gb, _L)] = jnp.where(lane == pos - gb, val_splat, grp)

        def iou_mask(bx1, by1, bx2, by2, ba, sl):
            xx1 = jnp.maximum(cx1[sl], bx1)
            yy1 = jnp.maximum(cy1[sl], by1)
            xx2 = jnp.minimum(cx2[sl], bx2)
            yy2 = jnp.minimum(cy2[sl], by2)
            inter = (jnp.maximum(xx2 - xx1, 0.0)
                     * jnp.maximum(yy2 - yy1, 0.0))
            iou = inter / (ba + car[sl] - inter + 1e-9)
            return iou > _NMS_THRESH

        def load_chunk(chunk):
            pltpu.sync_copy(order_hbm.at[pl.ds(chunk * _C, _C)], idx_v)
            cps = [pltpu.async_copy(h.at[idx_v], v, sem)
                   for h, v in ((x1h, cx1), (y1h, cy1), (x2h, cx2),
                                (y2h, cy2), (sch, csc), (lbh, clb))]
            for cp in cps:
                cp.wait()
            for g in range(_G):
                sl = pl.ds(g * _L, _L)
                car[sl] = (cx2[sl] - cx1[sl]) * (cy2[sl] - cy1[sl])
                alive_v[sl] = jnp.where(
                    (csc[sl] >= _SCORE_THRESH)
                    & (lane < n - (chunk * _C + g * _L)), 1, 0)

        def phase1(count):
            def body(k, _):
                gb = k & ~(_L - 1)
                lidx = k - gb
                sl = pl.ds(gb, _L)
                b1 = splat_at(kx1[sl], lidx)
                b2 = splat_at(ky1[sl], lidx)
                b3 = splat_at(kx2[sl], lidx)
                b4 = splat_at(ky2[sl], lidx)
                ba = splat_at(kar[sl], lidx)
                for g in range(_G):
                    gsl = pl.ds(g * _L, _L)
                    alive_v[gsl] = jnp.where(
                        iou_mask(b1, b2, b3, b4, ba, gsl), 0, alive_v[gsl])
                return 0
            lax.fori_loop(0, count, body, 0)

        def phase2(count):
            def body(p, cnt):
                gb = p & ~(_L - 1)
                lidx = p - gb
                sl = pl.ds(gb, _L)
                a = alive_v[pl.ds(p, _L)][0]

                def keep_it(cnt):
                    b1 = splat_at(cx1[sl], lidx)
                    b2 = splat_at(cy1[sl], lidx)
                    b3 = splat_at(cx2[sl], lidx)
                    b4 = splat_at(cy2[sl], lidx)
                    ba = splat_at(car[sl], lidx)
                    store_at(kx1, cnt, b1)
                    store_at(ky1, cnt, b2)
                    store_at(kx2, cnt, b3)
                    store_at(ky2, cnt, b4)
                    store_at(kar, cnt, ba)
                    store_at(ksc, cnt, splat_at(csc[sl], lidx))
                    store_at(klb, cnt, splat_at(clb[sl], lidx))
                    for g in range(_G):
                        gsl = pl.ds(g * _L, _L)
                        alive_v[gsl] = jnp.where(
                            iou_mask(b1, b2, b3, b4, ba, gsl),
                            0, alive_v[gsl])
                    return cnt + 1

                return lax.cond((a != 0) & (cnt < _K), keep_it,
                                lambda cnt: cnt, cnt)

            return lax.fori_loop(0, _C, body, count)

        def outer_body(chunk, carry):
            def process(carry):
                count, more = carry
                load_chunk(chunk)
                sc_top = csc[pl.ds(0, _L)][0]

                @pl.when((sc_top >= _SCORE_THRESH) & (count > 0))
                def _p1():
                    phase1(count)

                count = lax.cond(sc_top >= _SCORE_THRESH, phase2,
                                 lambda cnt: cnt, count)
                more = (csc[pl.ds(_C - _L, _L)][_L - 1]
                        >= _SCORE_THRESH).astype(jnp.int32)
                return (count, more)

            count, more = carry
            active = (more != 0) & (count < _K)
            return lax.cond(active, process, lambda c: c, carry)

        lax.fori_loop(0, n_chunks, outer_body,
                      (jnp.int32(0), jnp.int32(1)))

        pltpu.sync_copy(kx1, out_hbm.at[0])
        pltpu.sync_copy(ky1, out_hbm.at[1])
        pltpu.sync_copy(kx2, out_hbm.at[2])
        pltpu.sync_copy(ky2, out_hbm.at[3])
        pltpu.sync_copy(ksc, out_hbm.at[4])
        pltpu.sync_copy(klb, out_hbm.at[5])


def _build_nms():
    mesh = plsc.VectorSubcoreMesh(core_axis_name="c", subcore_axis_name="s")
    col = pltpu.VMEM((_C,), jnp.float32)
    return pl.kernel(
        _nms_body,
        out_type=jax.ShapeDtypeStruct((6, _C), jnp.float32),
        mesh=mesh,
        scratch_types=[
            pltpu.VMEM((_C,), jnp.int32),
            col, col, col, col, col, col,
            col,
            pltpu.VMEM((_C + _L,), jnp.int32),
            col, col, col, col, col, col,
            col,
            pltpu.SemaphoreType.DMA,
        ],
    )


@jax.jit
def kernel(boxes, scores, labels, top_k):
    n = boxes.shape[0]
    order = jnp.argsort(-scores).astype(jnp.int32)
    pad = (-n) % _C
    order_p = jnp.concatenate([order, jnp.zeros((pad,), jnp.int32)])
    bt = boxes.T
    lb = labels.astype(jnp.float32)

    rows = _build_nms()(bt[0], bt[1], bt[2], bt[3], scores, lb, order_p)

    out_scores = rows[4, :_K]
    valid = (out_scores > 0.0) & (jnp.arange(_K) < top_k)
    out_boxes = jnp.where(valid[:, None], rows[:4, :_K].T, 0.0)
    out_scores = jnp.where(valid, out_scores, 0.0)
    out_labels = jnp.where(valid, rows[5, :_K].astype(jnp.int32), -1)
    return out_boxes, out_scores, out_labels


# speedup vs baseline: 1.1725x; 1.0624x over previous
import jax
import jax.numpy as jnp
from jax import lax
from jax.experimental import pallas as pl
from jax.experimental.pallas import tpu as pltpu
from jax.experimental.pallas import tpu_sc as plsc

_L = 16
_C = 128
_G = _C // _L
_K = 100
_SCORE_THRESH = 0.5
_NMS_THRESH = 0.5


def _nms_body(x1h, y1h, x2h, y2h, sch, lbh, order_hbm, out_hbm,
              idx_v, cx1, cy1, cx2, cy2, csc, clb, car, alive_v,
              kx1, ky1, kx2, ky2, ksc, klb, kar, sem):
    n = sch.shape[0]
    n_pad = order_hbm.shape[0]
    n_chunks = n_pad // _C
    wid = lax.axis_index("s") * 2 + lax.axis_index("c")

    @pl.when(wid == 0)
    def _run():
        lane = lax.broadcasted_iota(jnp.int32, (_L,), 0)
        zero = jnp.zeros((_L,), jnp.float32)

        alive_v[pl.ds(_C, _L)] = jnp.zeros((_L,), jnp.int32)

        for g in range(_G):
            sl = pl.ds(g * _L, _L)
            kx1[sl] = zero
            ky1[sl] = zero
            kx2[sl] = zero
            ky2[sl] = zero
            ksc[sl] = zero
            klb[sl] = zero

        def splat_at(vec, j):
            return vec.at[jnp.full((_L,), j, jnp.int32)].get(
                mode="promise_in_bounds")

        def store_at(ref, pos, val_splat):
            gb = pos & ~(_L - 1)
            grp = ref[pl.ds(gb, _L)]
            ref[pl.ds(gb, _L)] = jnp.where(lane == pos - gb, val_splat, grp)

        def iou_mask(bx1, by1, bx2, by2, ba, sl):
            xx1 = jnp.maximum(cx1[sl], bx1)
            yy1 = jnp.maximum(cy1[sl], by1)
            xx2 = jnp.minimum(cx2[sl], bx2)
            yy2 = jnp.minimum(cy2[sl], by2)
            inter = (jnp.maximum(xx2 - xx1, 0.0)
                     * jnp.maximum(yy2 - yy1, 0.0))
            iou = inter / (ba + car[sl] - inter + 1e-9)
            return iou > _NMS_THRESH

        def load_chunk(chunk):
            pltpu.sync_copy(order_hbm.at[pl.ds(chunk * _C, _C)], idx_v)
            cps = [pltpu.async_copy(h.at[idx_v], v, sem)
                   for h, v in ((x1h, cx1), (y1h, cy1), (x2h, cx2),
                                (y2h, cy2), (sch, csc), (lbh, clb))]
            for cp in cps:
                cp.wait()
            for g in range(_G):
                sl = pl.ds(g * _L, _L)
                car[sl] = (cx2[sl] - cx1[sl]) * (cy2[sl] - cy1[sl])
                alive_v[sl] = jnp.where(
                    (csc[sl] >= _SCORE_THRESH)
                    & (lane < n - (chunk * _C + g * _L)), 1, 0)

        def phase1(count):
            def body(k, _):
                gb = k & ~(_L - 1)
                lidx = k - gb
                sl = pl.ds(gb, _L)
                b1 = splat_at(kx1[sl], lidx)
                b2 = splat_at(ky1[sl], lidx)
                b3 = splat_at(kx2[sl], lidx)
                b4 = splat_at(ky2[sl], lidx)
                ba = splat_at(kar[sl], lidx)
                for g in range(_G):
                    gsl = pl.ds(g * _L, _L)
                    alive_v[gsl] = jnp.where(
                        iou_mask(b1, b2, b3, b4, ba, gsl), 0, alive_v[gsl])
                return 0
            lax.fori_loop(0, count, body, 0)

        def phase2(count):
            def body(p, cnt):
                gb = p & ~(_L - 1)
                lidx = p - gb
                sl = pl.ds(gb, _L)
                a = alive_v[pl.ds(p, _L)][0]

                def keep_it(cnt):
                    b1 = splat_at(cx1[sl], lidx)
                    b2 = splat_at(cy1[sl], lidx)
                    b3 = splat_at(cx2[sl], lidx)
                    b4 = splat_at(cy2[sl], lidx)
                    ba = splat_at(car[sl], lidx)
                    store_at(kx1, cnt, b1)
                    store_at(ky1, cnt, b2)
                    store_at(kx2, cnt, b3)
                    store_at(ky2, cnt, b4)
                    store_at(kar, cnt, ba)
                    store_at(ksc, cnt, splat_at(csc[sl], lidx))
                    store_at(klb, cnt, splat_at(clb[sl], lidx))
                    for g in range(_G):
                        gsl = pl.ds(g * _L, _L)
                        alive_v[gsl] = jnp.where(
                            iou_mask(b1, b2, b3, b4, ba, gsl),
                            0, alive_v[gsl])
                    return cnt + 1

                return lax.cond((a != 0) & (cnt < _K), keep_it,
                                lambda cnt: cnt, cnt)

            return lax.fori_loop(0, _C, body, count)

        def outer_body(chunk, carry):
            def process(carry):
                count, more = carry
                load_chunk(chunk)
                sc_top = csc[pl.ds(0, _L)][0]

                @pl.when((sc_top >= _SCORE_THRESH) & (count > 0))
                def _p1():
                    phase1(count)

                count = lax.cond(sc_top >= _SCORE_THRESH, phase2,
                                 lambda cnt: cnt, count)
                more = (csc[pl.ds(_C - _L, _L)][_L - 1]
                        >= _SCORE_THRESH).astype(jnp.int32)
                return (count, more)

            count, more = carry
            active = (more != 0) & (count < _K)
            return lax.cond(active, process, lambda c: c, carry)

        lax.fori_loop(0, n_chunks, outer_body,
                      (jnp.int32(0), jnp.int32(1)))

        pltpu.sync_copy(kx1, out_hbm.at[0])
        pltpu.sync_copy(ky1, out_hbm.at[1])
        pltpu.sync_copy(kx2, out_hbm.at[2])
        pltpu.sync_copy(ky2, out_hbm.at[3])
        pltpu.sync_copy(ksc, out_hbm.at[4])
        pltpu.sync_copy(klb, out_hbm.at[5])


def _build_nms():
    mesh = plsc.VectorSubcoreMesh(core_axis_name="c", subcore_axis_name="s")
    col = pltpu.VMEM((_C,), jnp.float32)
    return pl.kernel(
        _nms_body,
        out_type=jax.ShapeDtypeStruct((6, _C), jnp.float32),
        mesh=mesh,
        scratch_types=[
            pltpu.VMEM((_C,), jnp.int32),
            col, col, col, col, col, col,
            col,
            pltpu.VMEM((_C + _L,), jnp.int32),
            col, col, col, col, col, col,
            col,
            pltpu.SemaphoreType.DMA,
        ],
    )


@jax.jit
def kernel(boxes, scores, labels, top_k):
    n = boxes.shape[0]
    bt = boxes.T
    lb = labels.astype(jnp.float32)

    vals, idx_top = lax.top_k(scores, _C)
    rows_fast = _build_nms()(bt[0], bt[1], bt[2], bt[3], scores, lb,
                             idx_top.astype(jnp.int32))
    kept_cnt = jnp.sum((rows_fast[4] > 0.0).astype(jnp.int32))
    need_more = (kept_cnt < _K) & (vals[_C - 1] >= _SCORE_THRESH)

    def full_path():
        order = jnp.argsort(-scores).astype(jnp.int32)
        pad = (-n) % _C
        order_p = jnp.concatenate([order, jnp.zeros((pad,), jnp.int32)])
        return _build_nms()(bt[0], bt[1], bt[2], bt[3], scores, lb, order_p)

    rows = lax.cond(need_more, full_path, lambda: rows_fast)

    out_scores = rows[4, :_K]
    valid = (out_scores > 0.0) & (jnp.arange(_K) < top_k)
    out_boxes = jnp.where(valid[:, None], rows[:4, :_K].T, 0.0)
    out_scores = jnp.where(valid, out_scores, 0.0)
    out_labels = jnp.where(valid, rows[5, :_K].astype(jnp.int32), -1)
    return out_boxes, out_scores, out_labels


# speedup vs baseline: 1.2127x; 1.0343x over previous
import jax
import jax.numpy as jnp
from jax import lax
from jax.experimental import pallas as pl
from jax.experimental.pallas import tpu as pltpu
from jax.experimental.pallas import tpu_sc as plsc

_L = 16
_C = 128
_G = _C // _L
_K = 100
_SCORE_THRESH = 0.5
_NMS_THRESH = 0.5


def _nms_body(x1h, y1h, x2h, y2h, sch, lbh, order_hbm,
              out_hbm, outlb_hbm,
              idx_v, cx1, cy1, cx2, cy2, csc, clb, car, alive_v,
              kx1, ky1, kx2, ky2, ksc, klb, kar, sem):
    n = sch.shape[0]
    n_pad = order_hbm.shape[0]
    n_chunks = n_pad // _C
    wid = lax.axis_index("s") * 2 + lax.axis_index("c")

    @pl.when(wid == 0)
    def _run():
        lane = lax.broadcasted_iota(jnp.int32, (_L,), 0)
        zero = jnp.zeros((_L,), jnp.float32)

        alive_v[pl.ds(_C, _L)] = jnp.zeros((_L,), jnp.int32)

        for g in range(_G):
            sl = pl.ds(g * _L, _L)
            kx1[sl] = zero
            ky1[sl] = zero
            kx2[sl] = zero
            ky2[sl] = zero
            ksc[sl] = zero
            klb[sl] = jnp.zeros((_L,), jnp.int32)

        def splat_at(vec, j):
            return vec.at[jnp.full((_L,), j, jnp.int32)].get(
                mode="promise_in_bounds")

        def store_at(ref, pos, val_splat):
            gb = pos & ~(_L - 1)
            grp = ref[pl.ds(gb, _L)]
            ref[pl.ds(gb, _L)] = jnp.where(lane == pos - gb, val_splat, grp)

        def iou_mask(bx1, by1, bx2, by2, ba, sl):
            xx1 = jnp.maximum(cx1[sl], bx1)
            yy1 = jnp.maximum(cy1[sl], by1)
            xx2 = jnp.minimum(cx2[sl], bx2)
            yy2 = jnp.minimum(cy2[sl], by2)
            inter = (jnp.maximum(xx2 - xx1, 0.0)
                     * jnp.maximum(yy2 - yy1, 0.0))
            iou = inter / (ba + car[sl] - inter + 1e-9)
            return iou > _NMS_THRESH

        def load_chunk(chunk):
            pltpu.sync_copy(order_hbm.at[pl.ds(chunk * _C, _C)], idx_v)
            cps = [pltpu.async_copy(h.at[idx_v], v, sem)
                   for h, v in ((x1h, cx1), (y1h, cy1), (x2h, cx2),
                                (y2h, cy2), (sch, csc), (lbh, clb))]
            for cp in cps:
                cp.wait()
            for g in range(_G):
                sl = pl.ds(g * _L, _L)
                car[sl] = (cx2[sl] - cx1[sl]) * (cy2[sl] - cy1[sl])
                alive_v[sl] = jnp.where(
                    (csc[sl] >= _SCORE_THRESH)
                    & (lane < n - (chunk * _C + g * _L)), 1, 0)

        def phase1(count):
            def body(k, _):
                gb = k & ~(_L - 1)
                lidx = k - gb
                sl = pl.ds(gb, _L)
                b1 = splat_at(kx1[sl], lidx)
                b2 = splat_at(ky1[sl], lidx)
                b3 = splat_at(kx2[sl], lidx)
                b4 = splat_at(ky2[sl], lidx)
                ba = splat_at(kar[sl], lidx)
                for g in range(_G):
                    gsl = pl.ds(g * _L, _L)
                    alive_v[gsl] = jnp.where(
                        iou_mask(b1, b2, b3, b4, ba, gsl), 0, alive_v[gsl])
                return 0
            lax.fori_loop(0, count, body, 0)

        def phase2(count):
            def body(p, cnt):
                gb = p & ~(_L - 1)
                lidx = p - gb
                sl = pl.ds(gb, _L)
                a = alive_v[pl.ds(p, _L)][0]

                def keep_it(cnt):
                    b1 = splat_at(cx1[sl], lidx)
                    b2 = splat_at(cy1[sl], lidx)
                    b3 = splat_at(cx2[sl], lidx)
                    b4 = splat_at(cy2[sl], lidx)
                    ba = splat_at(car[sl], lidx)
                    store_at(kx1, cnt, b1)
                    store_at(ky1, cnt, b2)
                    store_at(kx2, cnt, b3)
                    store_at(ky2, cnt, b4)
                    store_at(kar, cnt, ba)
                    store_at(ksc, cnt, splat_at(csc[sl], lidx))
                    store_at(klb, cnt, splat_at(clb[sl], lidx))
                    for g in range(_G):
                        gsl = pl.ds(g * _L, _L)
                        alive_v[gsl] = jnp.where(
                            iou_mask(b1, b2, b3, b4, ba, gsl),
                            0, alive_v[gsl])
                    return cnt + 1

                return lax.cond((a != 0) & (cnt < _K), keep_it,
                                lambda cnt: cnt, cnt)

            return lax.fori_loop(0, _C, body, count)

        def outer_body(chunk, carry):
            def process(carry):
                count, more = carry
                load_chunk(chunk)
                sc_top = csc[pl.ds(0, _L)][0]

                @pl.when((sc_top >= _SCORE_THRESH) & (count > 0))
                def _p1():
                    phase1(count)

                count = lax.cond(sc_top >= _SCORE_THRESH, phase2,
                                 lambda cnt: cnt, count)
                more = (csc[pl.ds(_C - _L, _L)][_L - 1]
                        >= _SCORE_THRESH).astype(jnp.int32)
                return (count, more)

            count, more = carry
            active = (more != 0) & (count < _K)
            return lax.cond(active, process, lambda c: c, carry)

        count, _ = lax.fori_loop(0, n_chunks, outer_body,
                                 (jnp.int32(0), jnp.int32(1)))

        cnt_f = count.astype(jnp.float32)
        for g in range(_G):
            car[pl.ds(g * _L, _L)] = jnp.full((_L,), cnt_f, jnp.float32)
        pltpu.sync_copy(kx1, out_hbm.at[0])
        pltpu.sync_copy(ky1, out_hbm.at[1])
        pltpu.sync_copy(kx2, out_hbm.at[2])
        pltpu.sync_copy(ky2, out_hbm.at[3])
        pltpu.sync_copy(ksc, out_hbm.at[4])
        pltpu.sync_copy(car, out_hbm.at[5])
        pltpu.sync_copy(klb, outlb_hbm)


def _build_nms():
    mesh = plsc.VectorSubcoreMesh(core_axis_name="c", subcore_axis_name="s")
    col = pltpu.VMEM((_C,), jnp.float32)
    coli = pltpu.VMEM((_C,), jnp.int32)
    return pl.kernel(
        _nms_body,
        out_type=(jax.ShapeDtypeStruct((6, _C), jnp.float32),
                  jax.ShapeDtypeStruct((_C,), jnp.int32)),
        mesh=mesh,
        scratch_types=[
            pltpu.VMEM((_C,), jnp.int32),
            col, col, col, col, col,
            coli,
            col,
            pltpu.VMEM((_C + _L,), jnp.int32),
            col, col, col, col, col,
            coli,
            col,
            pltpu.SemaphoreType.DMA,
        ],
    )


@jax.jit
def kernel(boxes, scores, labels, top_k):
    n = boxes.shape[0]
    bt = boxes.T
    lb = labels.astype(jnp.int32)

    vals, idx_top = lax.top_k(scores, _C)
    fast = _build_nms()(bt[0], bt[1], bt[2], bt[3], scores, lb,
                        idx_top.astype(jnp.int32))
    need_more = (fast[0][5, 0] < _K) & (vals[_C - 1] >= _SCORE_THRESH)

    def full_path():
        order = jnp.argsort(-scores).astype(jnp.int32)
        pad = (-n) % _C
        order_p = jnp.concatenate([order, jnp.zeros((pad,), jnp.int32)])
        return _build_nms()(bt[0], bt[1], bt[2], bt[3], scores, lb, order_p)

    rows, lbv = lax.cond(need_more, full_path, lambda: fast)

    out_scores = rows[4, :_K]
    valid = (out_scores > 0.0) & (jnp.arange(_K) < top_k)
    out_boxes = jnp.where(valid[:, None], rows[:4, :_K].T, 0.0)
    out_scores = jnp.where(valid, out_scores, 0.0)
    out_labels = jnp.where(valid, lbv[:_K], -1)
    return out_boxes, out_scores, out_labels


# speedup vs baseline: 1.2402x; 1.0227x over previous
import jax
import jax.numpy as jnp
from jax import lax
from jax.experimental import pallas as pl
from jax.experimental.pallas import tpu as pltpu
from jax.experimental.pallas import tpu_sc as plsc

_L = 16
_C = 128
_G = _C // _L
_K = 100
_SCORE_THRESH = 0.5
_NMS_THRESH = 0.5


def _nms_body(x1h, y1h, x2h, y2h, sch, lbh, order_hbm,
              out_hbm, outlb_hbm,
              idx_v, cx1, cy1, cx2, cy2, csc, clb, car, alive_v,
              kx1, ky1, kx2, ky2, ksc, klb, kar, sem):
    n = sch.shape[0]
    n_pad = order_hbm.shape[0]
    n_chunks = n_pad // _C
    wid = lax.axis_index("s") * 2 + lax.axis_index("c")

    @pl.when(wid == 0)
    def _run():
        lane = lax.broadcasted_iota(jnp.int32, (_L,), 0)
        zero = jnp.zeros((_L,), jnp.float32)

        alive_v[pl.ds(_C, _L)] = jnp.zeros((_L,), jnp.int32)

        for g in range(_G):
            sl = pl.ds(g * _L, _L)
            kx1[sl] = zero
            ky1[sl] = zero
            kx2[sl] = zero
            ky2[sl] = zero
            ksc[sl] = zero
            klb[sl] = jnp.zeros((_L,), jnp.int32)

        def splat_at(vec, j):
            return vec.at[jnp.full((_L,), j, jnp.int32)].get(
                mode="promise_in_bounds")

        def store_at(ref, pos, val_splat):
            gb = pos & ~(_L - 1)
            grp = ref[pl.ds(gb, _L)]
            ref[pl.ds(gb, _L)] = jnp.where(lane == pos - gb, val_splat, grp)

        def iou_mask(bx1, by1, bx2, by2, ba, sl):
            xx1 = jnp.maximum(cx1[sl], bx1)
            yy1 = jnp.maximum(cy1[sl], by1)
            xx2 = jnp.minimum(cx2[sl], bx2)
            yy2 = jnp.minimum(cy2[sl], by2)
            inter = (jnp.maximum(xx2 - xx1, 0.0)
                     * jnp.maximum(yy2 - yy1, 0.0))
            iou = inter / (ba + car[sl] - inter + 1e-9)
            return iou > _NMS_THRESH

        def load_chunk(chunk):
            pltpu.sync_copy(order_hbm.at[pl.ds(chunk * _C, _C)], idx_v)
            cps = [pltpu.async_copy(h.at[idx_v], v, sem)
                   for h, v in ((x1h, cx1), (y1h, cy1), (x2h, cx2),
                                (y2h, cy2), (sch, csc), (lbh, clb))]
            for cp in cps:
                cp.wait()
            for g in range(_G):
                sl = pl.ds(g * _L, _L)
                car[sl] = (cx2[sl] - cx1[sl]) * (cy2[sl] - cy1[sl])
                alive_v[sl] = jnp.where(
                    (csc[sl] >= _SCORE_THRESH)
                    & (lane < n - (chunk * _C + g * _L)), 1, 0)

        def phase1(count):
            def body(k, _):
                gb = k & ~(_L - 1)
                lidx = k - gb
                sl = pl.ds(gb, _L)
                b1 = splat_at(kx1[sl], lidx)
                b2 = splat_at(ky1[sl], lidx)
                b3 = splat_at(kx2[sl], lidx)
                b4 = splat_at(ky2[sl], lidx)
                ba = splat_at(kar[sl], lidx)
                for g in range(_G):
                    gsl = pl.ds(g * _L, _L)
                    alive_v[gsl] = jnp.where(
                        iou_mask(b1, b2, b3, b4, ba, gsl), 0, alive_v[gsl])
                return 0
            lax.fori_loop(0, count, body, 0)

        def phase2(count):
            def body(p, cnt):
                gb = p & ~(_L - 1)
                lidx = p - gb
                sl = pl.ds(gb, _L)
                a = alive_v[pl.ds(p, _L)][0]

                def keep_it(cnt):
                    b1 = splat_at(cx1[sl], lidx)
                    b2 = splat_at(cy1[sl], lidx)
                    b3 = splat_at(cx2[sl], lidx)
                    b4 = splat_at(cy2[sl], lidx)
                    ba = splat_at(car[sl], lidx)
                    store_at(kx1, cnt, b1)
                    store_at(ky1, cnt, b2)
                    store_at(kx2, cnt, b3)
                    store_at(ky2, cnt, b4)
                    store_at(kar, cnt, ba)
                    store_at(ksc, cnt, splat_at(csc[sl], lidx))
                    store_at(klb, cnt, splat_at(clb[sl], lidx))
                    for g in range(_G):
                        gsl = pl.ds(g * _L, _L)
                        alive_v[gsl] = jnp.where(
                            iou_mask(b1, b2, b3, b4, ba, gsl),
                            0, alive_v[gsl])
                    return cnt + 1

                return lax.cond((a != 0) & (cnt < _K), keep_it,
                                lambda cnt: cnt, cnt)

            return lax.fori_loop(0, _C, body, count)

        def outer_body(chunk, carry):
            def process(carry):
                count, more = carry
                load_chunk(chunk)
                sc_top = csc[pl.ds(0, _L)][0]

                @pl.when((sc_top >= _SCORE_THRESH) & (count > 0))
                def _p1():
                    phase1(count)

                count = lax.cond(sc_top >= _SCORE_THRESH, phase2,
                                 lambda cnt: cnt, count)
                more = (csc[pl.ds(_C - _L, _L)][_L - 1]
                        >= _SCORE_THRESH).astype(jnp.int32)
                return (count, more)

            count, more = carry
            active = (more != 0) & (count < _K)
            return lax.cond(active, process, lambda c: c, carry)

        count, _ = lax.fori_loop(0, n_chunks, outer_body,
                                 (jnp.int32(0), jnp.int32(1)))

        cnt_f = count.astype(jnp.float32)
        for g in range(_G):
            car[pl.ds(g * _L, _L)] = jnp.full((_L,), cnt_f, jnp.float32)
        pltpu.sync_copy(kx1, out_hbm.at[0])
        pltpu.sync_copy(ky1, out_hbm.at[1])
        pltpu.sync_copy(kx2, out_hbm.at[2])
        pltpu.sync_copy(ky2, out_hbm.at[3])
        pltpu.sync_copy(ksc, out_hbm.at[4])
        pltpu.sync_copy(car, out_hbm.at[5])
        pltpu.sync_copy(klb, outlb_hbm)


def _build_nms():
    mesh = plsc.VectorSubcoreMesh(core_axis_name="c", subcore_axis_name="s",
                                  num_cores=1)
    col = pltpu.VMEM((_C,), jnp.float32)
    coli = pltpu.VMEM((_C,), jnp.int32)
    return pl.kernel(
        _nms_body,
        out_type=(jax.ShapeDtypeStruct((6, _C), jnp.float32),
                  jax.ShapeDtypeStruct((_C,), jnp.int32)),
        mesh=mesh,
        scratch_types=[
            pltpu.VMEM((_C,), jnp.int32),
            col, col, col, col, col,
            coli,
            col,
            pltpu.VMEM((_C + _L,), jnp.int32),
            col, col, col, col, col,
            coli,
            col,
            pltpu.SemaphoreType.DMA,
        ],
    )


@jax.jit
def kernel(boxes, scores, labels, top_k):
    n = boxes.shape[0]
    bt = boxes.T
    lb = labels.astype(jnp.int32)

    vals, idx_top = lax.top_k(scores, _C)
    fast = _build_nms()(bt[0], bt[1], bt[2], bt[3], scores, lb,
                        idx_top.astype(jnp.int32))
    need_more = (fast[0][5, 0] < _K) & (vals[_C - 1] >= _SCORE_THRESH)

    def full_path():
        order = jnp.argsort(-scores).astype(jnp.int32)
        pad = (-n) % _C
        order_p = jnp.concatenate([order, jnp.zeros((pad,), jnp.int32)])
        return _build_nms()(bt[0], bt[1], bt[2], bt[3], scores, lb, order_p)

    rows, lbv = lax.cond(need_more, full_path, lambda: fast)

    out_scores = rows[4, :_K]
    valid = (out_scores > 0.0) & (jnp.arange(_K) < top_k)
    out_boxes = jnp.where(valid[:, None], rows[:4, :_K].T, 0.0)
    out_scores = jnp.where(valid, out_scores, 0.0)
    out_labels = jnp.where(valid, lbv[:_K], -1)
    return out_boxes, out_scores, out_labels


# speedup vs baseline: 1.3265x; 1.0696x over previous
import jax
import jax.numpy as jnp
from jax import lax
from jax.experimental import pallas as pl
from jax.experimental.pallas import tpu as pltpu
from jax.experimental.pallas import tpu_sc as plsc

_L = 16
_C = 128
_G = _C // _L
_K = 100
_SCORE_THRESH = 0.5
_NMS_THRESH = 0.5


def _nms_body(x1h, y1h, x2h, y2h, sch, lbh, order_hbm,
              out_hbm, outlb_hbm,
              idx_v, cx1, cy1, cx2, cy2, csc, clb, car, alive_v,
              kx1, ky1, kx2, ky2, ksc, klb, kar, sem):
    n = sch.shape[0]
    n_pad = order_hbm.shape[0]
    n_chunks = n_pad // _C
    wid = lax.axis_index("s") * 2 + lax.axis_index("c")

    @pl.when(wid == 0)
    def _run():
        lane = lax.broadcasted_iota(jnp.int32, (_L,), 0)
        zero = jnp.zeros((_L,), jnp.float32)

        alive_v[pl.ds(_C, _L)] = jnp.zeros((_L,), jnp.int32)

        for g in range(_G):
            sl = pl.ds(g * _L, _L)
            kx1[sl] = zero
            ky1[sl] = zero
            kx2[sl] = zero
            ky2[sl] = zero
            ksc[sl] = zero
            klb[sl] = jnp.zeros((_L,), jnp.int32)

        def splat_at(vec, j):
            return vec.at[jnp.full((_L,), j, jnp.int32)].get(
                mode="promise_in_bounds")

        def store_at(ref, pos, val_splat):
            gb = pos & ~(_L - 1)
            grp = ref[pl.ds(gb, _L)]
            ref[pl.ds(gb, _L)] = jnp.where(lane == pos - gb, val_splat, grp)

        def iou_mask(bx1, by1, bx2, by2, ba, sl):
            xx1 = jnp.maximum(cx1[sl], bx1)
            yy1 = jnp.maximum(cy1[sl], by1)
            xx2 = jnp.minimum(cx2[sl], bx2)
            yy2 = jnp.minimum(cy2[sl], by2)
            inter = (jnp.maximum(xx2 - xx1, 0.0)
                     * jnp.maximum(yy2 - yy1, 0.0))
            iou = inter / (ba + car[sl] - inter + 1e-9)
            return iou > _NMS_THRESH

        def load_chunk(chunk):
            pltpu.sync_copy(order_hbm.at[pl.ds(chunk * _C, _C)], idx_v)
            cps = [pltpu.async_copy(h.at[idx_v], v, sem)
                   for h, v in ((x1h, cx1), (y1h, cy1), (x2h, cx2),
                                (y2h, cy2), (sch, csc), (lbh, clb))]
            for cp in cps:
                cp.wait()
            for g in range(_G):
                sl = pl.ds(g * _L, _L)
                car[sl] = (cx2[sl] - cx1[sl]) * (cy2[sl] - cy1[sl])
                alive_v[sl] = jnp.where(
                    (csc[sl] >= _SCORE_THRESH)
                    & (lane < n - (chunk * _C + g * _L)), 1, 0)

        def phase1(count):
            def body(k, _):
                gb = k & ~(_L - 1)
                lidx = k - gb
                sl = pl.ds(gb, _L)
                b1 = splat_at(kx1[sl], lidx)
                b2 = splat_at(ky1[sl], lidx)
                b3 = splat_at(kx2[sl], lidx)
                b4 = splat_at(ky2[sl], lidx)
                ba = splat_at(kar[sl], lidx)
                for g in range(_G):
                    gsl = pl.ds(g * _L, _L)
                    alive_v[gsl] = jnp.where(
                        iou_mask(b1, b2, b3, b4, ba, gsl), 0, alive_v[gsl])
                return 0
            lax.fori_loop(0, count, body, 0)

        def phase2(count):
            gx1 = [cx1[pl.ds(g * _L, _L)] for g in range(_G)]
            gy1 = [cy1[pl.ds(g * _L, _L)] for g in range(_G)]
            gx2 = [cx2[pl.ds(g * _L, _L)] for g in range(_G)]
            gy2 = [cy2[pl.ds(g * _L, _L)] for g in range(_G)]
            gar = [car[pl.ds(g * _L, _L)] for g in range(_G)]

            def body(p, cnt):
                gb = p & ~(_L - 1)
                lidx = p - gb
                sl = pl.ds(gb, _L)
                a = alive_v[pl.ds(p, _L)][0]

                def keep_it(cnt):
                    b1 = splat_at(cx1[sl], lidx)
                    b2 = splat_at(cy1[sl], lidx)
                    b3 = splat_at(cx2[sl], lidx)
                    b4 = splat_at(cy2[sl], lidx)
                    ba = splat_at(car[sl], lidx)
                    store_at(kx1, cnt, b1)
                    store_at(ky1, cnt, b2)
                    store_at(kx2, cnt, b3)
                    store_at(ky2, cnt, b4)
                    store_at(kar, cnt, ba)
                    store_at(ksc, cnt, splat_at(csc[sl], lidx))
                    store_at(klb, cnt, splat_at(clb[sl], lidx))
                    for g in range(_G):
                        gsl = pl.ds(g * _L, _L)
                        xx1 = jnp.maximum(gx1[g], b1)
                        yy1 = jnp.maximum(gy1[g], b2)
                        xx2 = jnp.minimum(gx2[g], b3)
                        yy2 = jnp.minimum(gy2[g], b4)
                        inter = (jnp.maximum(xx2 - xx1, 0.0)
                                 * jnp.maximum(yy2 - yy1, 0.0))
                        iou = inter / (ba + gar[g] - inter + 1e-9)
                        alive_v[gsl] = jnp.where(
                            iou > _NMS_THRESH, 0, alive_v[gsl])
                    return cnt + 1

                return lax.cond((a != 0) & (cnt < _K), keep_it,
                                lambda cnt: cnt, cnt)

            return lax.fori_loop(0, _C, body, count)

        def outer_body(chunk, carry):
            def process(carry):
                count, more = carry
                load_chunk(chunk)
                sc_top = csc[pl.ds(0, _L)][0]

                @pl.when((sc_top >= _SCORE_THRESH) & (count > 0))
                def _p1():
                    phase1(count)

                count = lax.cond(sc_top >= _SCORE_THRESH, phase2,
                                 lambda cnt: cnt, count)
                more = (csc[pl.ds(_C - _L, _L)][_L - 1]
                        >= _SCORE_THRESH).astype(jnp.int32)
                return (count, more)

            count, more = carry
            active = (more != 0) & (count < _K)
            return lax.cond(active, process, lambda c: c, carry)

        count, _ = lax.fori_loop(0, n_chunks, outer_body,
                                 (jnp.int32(0), jnp.int32(1)))

        cnt_f = count.astype(jnp.float32)
        for g in range(_G):
            car[pl.ds(g * _L, _L)] = jnp.full((_L,), cnt_f, jnp.float32)
        pltpu.sync_copy(kx1, out_hbm.at[0])
        pltpu.sync_copy(ky1, out_hbm.at[1])
        pltpu.sync_copy(kx2, out_hbm.at[2])
        pltpu.sync_copy(ky2, out_hbm.at[3])
        pltpu.sync_copy(ksc, out_hbm.at[4])
        pltpu.sync_copy(car, out_hbm.at[5])
        pltpu.sync_copy(klb, outlb_hbm)


def _build_nms():
    mesh = plsc.VectorSubcoreMesh(core_axis_name="c", subcore_axis_name="s",
                                  num_cores=1)
    col = pltpu.VMEM((_C,), jnp.float32)
    coli = pltpu.VMEM((_C,), jnp.int32)
    return pl.kernel(
        _nms_body,
        out_type=(jax.ShapeDtypeStruct((6, _C), jnp.float32),
                  jax.ShapeDtypeStruct((_C,), jnp.int32)),
        mesh=mesh,
        scratch_types=[
            pltpu.VMEM((_C,), jnp.int32),
            col, col, col, col, col,
            coli,
            col,
            pltpu.VMEM((_C + _L,), jnp.int32),
            col, col, col, col, col,
            coli,
            col,
            pltpu.SemaphoreType.DMA,
        ],
    )


@jax.jit
def kernel(boxes, scores, labels, top_k):
    n = boxes.shape[0]
    bt = boxes.T
    lb = labels.astype(jnp.int32)

    vals, idx_top = lax.top_k(scores, _C)
    fast = _build_nms()(bt[0], bt[1], bt[2], bt[3], scores, lb,
                        idx_top.astype(jnp.int32))
    need_more = (fast[0][5, 0] < _K) & (vals[_C - 1] >= _SCORE_THRESH)

    def full_path():
        order = jnp.argsort(-scores).astype(jnp.int32)
        pad = (-n) % _C
        order_p = jnp.concatenate([order, jnp.zeros((pad,), jnp.int32)])
        return _build_nms()(bt[0], bt[1], bt[2], bt[3], scores, lb, order_p)

    rows, lbv = lax.cond(need_more, full_path, lambda: fast)

    out_scores = rows[4, :_K]
    valid = (out_scores > 0.0) & (jnp.arange(_K) < top_k)
    out_boxes = jnp.where(valid[:, None], rows[:4, :_K].T, 0.0)
    out_scores = jnp.where(valid, out_scores, 0.0)
    out_labels = jnp.where(valid, lbv[:_K], -1)
    return out_boxes, out_scores, out_labels
